# baseline (speedup 1.0000x reference)
import functools

import jax
import jax.numpy as jnp
from jax import lax
from jax.experimental import pallas as pl
from jax.experimental.pallas import tpu as pltpu
import jax.experimental.pallas.tpu_sc as plsc

B = 256
C = 128
NK = C // 16
STEPS = 8
NWORK = 32
SEG_PER_W = B // NWORK
CHUNK = 256
GR = 4


def _tree(op, vals):
    while len(vals) > 1:
        vals = [op(vals[i], vals[i + 1]) if i + 1 < len(vals) else vals[i]
                for i in range(0, len(vals), 2)]
    return vals[0]
NEG = -3.0e38


def _lstm_tc(h, r, c, m_w, bias):
    def body(h_ref, r_ref, c_ref, m_ref, b_ref, h_out, c_out):
        hr = jnp.concatenate([h_ref[...], r_ref[...]], axis=-1)
        gates = jnp.dot(hr, m_ref[...], preferred_element_type=jnp.float32)
        gates = gates + b_ref[...]
        i = jax.nn.sigmoid(gates[:, 0 * C:1 * C])
        f = jax.nn.sigmoid(gates[:, 1 * C:2 * C])
        g = jnp.tanh(gates[:, 2 * C:3 * C])
        o = jax.nn.sigmoid(gates[:, 3 * C:4 * C])
        c_new = f * c_ref[...] + i * g
        c_out[...] = c_new
        h_out[...] = o * jnp.tanh(c_new)

    return pl.pallas_call(
        body,
        out_shape=[jax.ShapeDtypeStruct((B, C), jnp.float32),
                   jax.ShapeDtypeStruct((B, C), jnp.float32)],
    )(h, r, c, m_w, bias)


def _make_attn(n_rows):
    assert n_rows % 8 == 0 and n_rows > CHUNK
    mesh = plsc.VectorSubcoreMesh(core_axis_name="c", subcore_axis_name="s")
    clamp_max = ((n_rows - CHUNK) // 8) * 8

    @functools.partial(
        pl.kernel,
        out_type=jax.ShapeDtypeStruct((B, C), jnp.float32),
        mesh=mesh,
        scratch_types=[
            pltpu.VMEM((16, 16), jnp.int32),
            pltpu.VMEM((SEG_PER_W, C), jnp.float32),
            pltpu.VMEM((CHUNK, C), jnp.float32),
            pltpu.VMEM((CHUNK, C), jnp.float32),
            pltpu.VMEM((SEG_PER_W, C), jnp.float32),
            pltpu.VMEM((SEG_PER_W, C), jnp.float32),
            pltpu.VMEM((SEG_PER_W, 16), jnp.float32),
            pltpu.VMEM((SEG_PER_W, 16), jnp.float32),
            pltpu.SemaphoreType.DMA,
            pltpu.SemaphoreType.DMA,
        ],
        compiler_params=pltpu.CompilerParams(needs_layout_passes=False),
    )
    def attn(x_hbm, q_hbm, offs_hbm, out_hbm,
             offs_v, q_v, xba, xbb, rstate, rbuf, msv, ssv, sema, semb):
        wid = lax.axis_index("s") * 2 + lax.axis_index("c")
        seg0 = wid * SEG_PER_W
        pltpu.sync_copy(offs_hbm.at[pl.ds(seg0, 16)], offs_v)
        pltpu.sync_copy(q_hbm.at[pl.ds(seg0, SEG_PER_W)], q_v)
        zero16 = jnp.zeros((16,), jnp.float32)
        neg16 = jnp.full((16,), NEG, jnp.float32)
        for j in range(SEG_PER_W):
            msv[j, pl.ds(0, 16)] = neg16
            ssv[j, pl.ds(0, 16)] = zero16
            for k in range(NK):
                rstate[j, pl.ds(16 * k, 16)] = zero16

        row_lo = offs_v[0, pl.ds(0, 16)][0]
        row_hi = offs_v[SEG_PER_W, pl.ds(0, 16)][0]
        a_lo = (row_lo // 8) * 8
        nch = (row_hi - a_lo + (CHUNK - 1)) // CHUNK
        npair = (nch + 1) // 2

        def chunk_slice(cid):
            base = a_lo + cid * CHUNK
            base_cl = jnp.minimum(base, clamp_max)
            return base, x_hbm.at[pl.ds(base_cl, CHUNK)]

        def process(xbuf, cid):
            base, _ = chunk_slice(cid)
            base_cl = jnp.minimum(base, clamp_max)

            def seg_body(j, _):
                lo = jnp.maximum(offs_v[j, pl.ds(0, 16)][0], base)
                hi = jnp.minimum(offs_v[j + 1, pl.ds(0, 16)][0], base + CHUNK)
                nrows = jnp.maximum(hi - lo, 0)

                @pl.when(nrows > 0)
                def _():
                    nfull = nrows // GR
                    trip = (nrows + GR - 1) // GR
                    qj = [q_v[j, pl.ds(16 * k, 16)] for k in range(NK)]
                    m = msv[j, pl.ds(0, 16)]
                    s = ssv[j, pl.ds(0, 16)]
                    r = [rstate[j, pl.ds(16 * k, 16)] for k in range(NK)]

                    def full_body(g, carry):
                        m, s, r = carry
                        row0 = lo + GR * g - base_cl
                        es = []
                        for d in range(GR):
                            acc = xbuf[row0 + d, pl.ds(0, 16)] * qj[0]
                            for k in range(1, NK):
                                acc = acc + (xbuf[row0 + d, pl.ds(16 * k, 16)]
                                             * qj[k])
                            es.append(
                                jnp.broadcast_to(jnp.sum(acc, axis=0), (16,)))
                        m_new = jnp.maximum(m, _tree(jnp.maximum, list(es)))
                        alpha = jnp.exp(m - m_new)
                        p = [jnp.exp(es[d] - m_new) for d in range(GR)]
                        s_new = s * alpha + _tree(jnp.add, list(p))
                        r_new = []
                        for k in range(NK):
                            acc = r[k] * alpha
                            for d in range(GR):
                                acc = acc + (p[d]
                                             * xbuf[row0 + d, pl.ds(16 * k, 16)])
                            r_new.append(acc)
                        return m_new, s_new, r_new

                    def tail_body(g, carry):
                        m, s, r = carry
                        row0 = lo + GR * g
                        es, valids, ridxs = [], [], []
                        for d in range(GR):
                            grow = row0 + d
                            valid = grow < hi
                            ridx = jnp.minimum(grow - base_cl, CHUNK - 1)
                            ridxs.append(ridx)
                            valids.append(valid)
                            acc = xbuf[ridx, pl.ds(0, 16)] * qj[0]
                            for k in range(1, NK):
                                acc = acc + (xbuf[ridx, pl.ds(16 * k, 16)]
                                             * qj[k])
                            es.append(
                                jnp.broadcast_to(jnp.sum(acc, axis=0), (16,)))
                        eeff = [jnp.where(valids[d], es[d], NEG)
                                for d in range(GR)]
                        m_new = jnp.maximum(m, _tree(jnp.maximum, eeff))
                        alpha = jnp.exp(m - m_new)
                        p = [jnp.where(valids[d], jnp.exp(es[d] - m_new),
                                       zero16) for d in range(GR)]
                        s_new = s * alpha + _tree(jnp.add, list(p))
                        r_new = []
                        for k in range(NK):
                            acc = r[k] * alpha
                            for d in range(GR):
                                acc = acc + (p[d]
                                             * xbuf[ridxs[d],
                                                    pl.ds(16 * k, 16)])
                            r_new.append(acc)
                        return m_new, s_new, r_new

                    m2, s2, r2 = lax.fori_loop(0, nfull, full_body, (m, s, r))
                    m2, s2, r2 = lax.fori_loop(nfull, trip, tail_body,
                                               (m2, s2, r2))
                    msv[j, pl.ds(0, 16)] = m2
                    ssv[j, pl.ds(0, 16)] = s2
                    for k in range(NK):
                        rstate[j, pl.ds(16 * k, 16)] = r2[k]

                return 0

            lax.fori_loop(0, SEG_PER_W, seg_body, 0)

        def pair_body(i, carry):
            c0 = 2 * i
            _, src0 = chunk_slice(c0)
            pltpu.make_async_copy(src0, xba, sema).wait()

            @pl.when(c0 + 1 < nch)
            def _():
                _, src1 = chunk_slice(c0 + 1)
                pltpu.async_copy(src1, xbb, semb)

            process(xba, c0)

            @pl.when(c0 + 1 < nch)
            def _():
                _, src1 = chunk_slice(c0 + 1)
                pltpu.make_async_copy(src1, xbb, semb).wait()

            @pl.when(c0 + 2 < nch)
            def _():
                _, src2 = chunk_slice(c0 + 2)
                pltpu.async_copy(src2, xba, sema)

            process(xbb, c0 + 1)
            return 0

        @pl.when(nch > 0)
        def _():
            _, src0 = chunk_slice(0)
            pltpu.async_copy(src0, xba, sema)

        lax.fori_loop(0, npair, pair_body, 0)

        for j in range(SEG_PER_W):
            s = ssv[j, pl.ds(0, 16)]
            inv = 1.0 / (s + 1e-16)
            for k in range(NK):
                rbuf[j, pl.ds(16 * k, 16)] = rstate[j, pl.ds(16 * k, 16)] * inv
        pltpu.sync_copy(rbuf, out_hbm.at[pl.ds(seg0, SEG_PER_W)])

    return attn


def kernel(x, w_ih, w_hh, b_ih, b_hh, batch_index):
    n_rows = x.shape[0]
    m_w = jnp.concatenate([w_ih.T[:C] + w_hh.T, w_ih.T[C:]], axis=0)
    bias = (b_ih + b_hh)[None, :]
    offs = jnp.searchsorted(
        batch_index, jnp.arange(B + 1, dtype=jnp.int32)).astype(jnp.int32)
    offs = jnp.concatenate([offs, jnp.full((7,), n_rows, jnp.int32)])
    offs = jnp.broadcast_to(offs[:, None], (B + 8, 16))

    attn = _make_attn(n_rows)
    h = jnp.zeros((B, C), jnp.float32)
    c = jnp.zeros((B, C), jnp.float32)
    r = jnp.zeros((B, C), jnp.float32)
    for _ in range(STEPS):
        h, c = _lstm_tc(h, r, c, m_w, bias)
        r = attn(x, h, offs)
    return jnp.concatenate([h, r], axis=-1)
